# baseline (speedup 1.0000x reference)
import functools
import math

import jax
import jax.numpy as jnp
from jax import lax
from jax.experimental import pallas as pl
from jax.experimental.pallas import tpu as pltpu
from jax.experimental.pallas import tpu_sc as plsc

_V = 32000
_SMOOTHING = 0.1
_CONF = 1.0 - _SMOOTHING
_EPS = _SMOOTHING / _V

_TC_BLOCK = 256
_SC_ROWS = 3072
_LANES = 16
_CHUNK_COLS = 3200
_N_CHUNKS = _V // _CHUNK_COLS


def _loss_block_kernel(x_ref, tgt_ref, out_ref):
    x = x_ref[...]
    tgt = tgt_ref[0, 0, :]
    r, v = x.shape
    cols = jax.lax.broadcasted_iota(jnp.int32, (r, v), 1)
    hit = cols == tgt[:, None]
    gathered = jnp.sum(jnp.where(hit, x, 0.0))
    total = jnp.sum(x)
    partial = -_EPS * total - (_CONF - _EPS) * gathered
    out_ref[...] = partial.reshape(1, 1, 1)


def _tree_sum(vecs):
    while len(vecs) > 1:
        vecs = [a + b for a, b in zip(vecs[::2], vecs[1::2])]
    return vecs[0]


def _make_sc_sum(tc_rows):
    info = plsc.get_sparse_core_info()
    nc, ns = info.num_cores, info.num_subcores
    nw = nc * ns
    rpw = _SC_ROWS // nw
    groups = rpw // _LANES

    mesh = plsc.VectorSubcoreMesh(core_axis_name="c", subcore_axis_name="s")

    @functools.partial(
        pl.kernel,
        mesh=mesh,
        out_type=jax.ShapeDtypeStruct((nw * _LANES,), jnp.float32),
        scratch_types=[
            pltpu.VMEM((rpw,), jnp.int32),
            pltpu.VMEM((_LANES, _CHUNK_COLS), jnp.float32),
            pltpu.VMEM((_LANES,), jnp.float32),
        ],
    )
    def sc_sum(x_hbm, tgt_hbm, out_hbm, tgt_v, buf_v, part_v):
        wid = lax.axis_index("s") * nc + lax.axis_index("c")
        base = tc_rows + wid * rpw
        pltpu.sync_copy(tgt_hbm.at[pl.ds(base, rpw)], tgt_v)
        lanes = lax.iota(jnp.int32, _LANES)

        acc = jnp.zeros((_LANES,), jnp.float32)
        corr = jnp.zeros((_LANES,), jnp.float32)
        for g in range(groups):
            row0 = base + g * _LANES
            tvec = tgt_v[pl.ds(g * _LANES, _LANES)]
            ts = [tvec[r] for r in range(_LANES)]
            for c in range(_N_CHUNKS):
                c0 = c * _CHUNK_COLS
                pltpu.sync_copy(
                    x_hbm.at[pl.ds(row0, _LANES), pl.ds(c0, _CHUNK_COLS)],
                    buf_v,
                )

                def col_body(i, carry):
                    a, cr = carry
                    colids = (c0 + i * _LANES) + lanes
                    vecs = [
                        buf_v[r, pl.ds(i * _LANES, _LANES)]
                        for r in range(_LANES)
                    ]
                    hits = [
                        jnp.where(colids == ts[r], vecs[r], 0.0)
                        for r in range(_LANES)
                    ]
                    return a + _tree_sum(vecs), cr + _tree_sum(hits)

                acc, corr = lax.fori_loop(
                    0, _CHUNK_COLS // _LANES, col_body, (acc, corr))
        part_v[...] = -_EPS * acc - (_CONF - _EPS) * corr
        pltpu.sync_copy(part_v, out_hbm.at[pl.ds(wid * _LANES, _LANES)])

    return sc_sum


def kernel(x, target):
    n, v = x.shape
    tc_rows = n - _SC_ROWS
    g = tc_rows // _TC_BLOCK
    tgt = target.astype(jnp.int32)
    tgt3 = tgt[:tc_rows].reshape(g, 1, _TC_BLOCK)
    tc_partials = pl.pallas_call(
        _loss_block_kernel,
        grid=(g,),
        in_specs=[
            pl.BlockSpec((_TC_BLOCK, v), lambda i: (i, 0)),
            pl.BlockSpec((1, 1, _TC_BLOCK), lambda i: (i, 0, 0)),
        ],
        out_specs=pl.BlockSpec((1, 1, 1), lambda i: (i, 0, 0)),
        out_shape=jax.ShapeDtypeStruct((g, 1, 1), jnp.float32),
        compiler_params=pltpu.CompilerParams(
            dimension_semantics=("parallel",),
            vmem_limit_bytes=128 * 1024 * 1024,
        ),
    )(x, tgt3)
    sc_partials = _make_sc_sum(tc_rows)(x, tgt)
    const = n * ((v - 1) * _EPS * math.log(_EPS) + _CONF * math.log(_CONF))
    return jnp.float32(const) + jnp.sum(tc_partials) + jnp.sum(sc_partials)


# speedup vs baseline: 1.0924x; 1.0924x over previous
import functools
import math

import jax
import jax.numpy as jnp
from jax import lax
from jax.experimental import pallas as pl
from jax.experimental.pallas import tpu as pltpu
from jax.experimental.pallas import tpu_sc as plsc

_V = 32000
_SMOOTHING = 0.1
_CONF = 1.0 - _SMOOTHING
_EPS = _SMOOTHING / _V

_TC_BLOCK = 256
_SC_ROWS = 3072
_LANES = 16
_CHUNK_COLS = 3200
_N_CHUNKS = _V // _CHUNK_COLS


def _loss_block_kernel(x_ref, tgt_ref, out_ref):
    x = x_ref[...]
    tgt = tgt_ref[0, 0, :]
    r, v = x.shape
    cols = jax.lax.broadcasted_iota(jnp.int32, (r, v), 1)
    hit = cols == tgt[:, None]
    gathered = jnp.sum(jnp.where(hit, x, 0.0))
    total = jnp.sum(x)
    partial = -_EPS * total - (_CONF - _EPS) * gathered
    out_ref[...] = partial.reshape(1, 1, 1)


def _tree_sum(vecs):
    while len(vecs) > 1:
        vecs = [a + b for a, b in zip(vecs[::2], vecs[1::2])]
    return vecs[0]


def _make_sc_sum(tc_rows):
    info = plsc.get_sparse_core_info()
    nc, ns = info.num_cores, info.num_subcores
    nw = nc * ns
    rpw = _SC_ROWS // nw
    groups = rpw // _LANES

    mesh = plsc.VectorSubcoreMesh(core_axis_name="c", subcore_axis_name="s")

    @functools.partial(
        pl.kernel,
        mesh=mesh,
        out_type=jax.ShapeDtypeStruct((nw * _LANES,), jnp.float32),
        scratch_types=[
            pltpu.VMEM((rpw,), jnp.int32),
            pltpu.VMEM((_LANES, _CHUNK_COLS), jnp.float32),
            pltpu.VMEM((_LANES, _CHUNK_COLS), jnp.float32),
            pltpu.VMEM((_LANES,), jnp.float32),
            pltpu.SemaphoreType.DMA,
            pltpu.SemaphoreType.DMA,
        ],
    )
    def sc_sum(x_hbm, tgt_hbm, out_hbm, tgt_v, buf0_v, buf1_v, part_v,
               sem0, sem1):
        wid = lax.axis_index("s") * nc + lax.axis_index("c")
        base = tc_rows + wid * rpw
        pltpu.sync_copy(tgt_hbm.at[pl.ds(base, rpw)], tgt_v)
        lanes = lax.iota(jnp.int32, _LANES)
        n_chunks_total = groups * _N_CHUNKS
        bufs = (buf0_v, buf1_v)
        sems = (sem0, sem1)

        def chunk_slices(k):
            row0 = base + (k // _N_CHUNKS) * _LANES
            c0 = (k % _N_CHUNKS) * _CHUNK_COLS
            return row0, c0

        def dma(k, b):
            row0, c0 = chunk_slices(k)
            return pltpu.make_async_copy(
                x_hbm.at[pl.ds(row0, _LANES), pl.ds(c0, _CHUNK_COLS)],
                bufs[b], sems[b])

        def compute(k, b, acc, corr):
            c0 = (k % _N_CHUNKS) * _CHUNK_COLS
            g = k // _N_CHUNKS
            tvec = tgt_v[pl.ds(g * _LANES, _LANES)]
            ts = [tvec[r] for r in range(_LANES)]
            buf = bufs[b]

            def col_body(i, carry):
                a, cr = carry
                colids = (c0 + i * _LANES) + lanes
                vecs = [buf[r, pl.ds(i * _LANES, _LANES)]
                        for r in range(_LANES)]
                hits = [jnp.where(colids == ts[r], vecs[r], 0.0)
                        for r in range(_LANES)]
                return a + _tree_sum(vecs), cr + _tree_sum(hits)

            return lax.fori_loop(
                0, _CHUNK_COLS // _LANES, col_body, (acc, corr))

        dma(0, 0).start()
        dma(1, 1).start()

        def pair_body(i, carry):
            acc, corr = carry
            k = 2 * i
            dma(k, 0).wait()
            acc, corr = compute(k, 0, acc, corr)

            @pl.when(k + 2 < n_chunks_total)
            def _():
                dma(k + 2, 0).start()

            dma(k + 1, 1).wait()
            acc, corr = compute(k + 1, 1, acc, corr)

            @pl.when(k + 3 < n_chunks_total)
            def _():
                dma(k + 3, 1).start()

            return acc, corr

        zero = jnp.zeros((_LANES,), jnp.float32)
        acc, corr = lax.fori_loop(
            0, n_chunks_total // 2, pair_body, (zero, zero))
        part_v[...] = -_EPS * acc - (_CONF - _EPS) * corr
        pltpu.sync_copy(part_v, out_hbm.at[pl.ds(wid * _LANES, _LANES)])

    return sc_sum


def kernel(x, target):
    n, v = x.shape
    tc_rows = n - _SC_ROWS
    g = tc_rows // _TC_BLOCK
    tgt = target.astype(jnp.int32)
    tgt3 = tgt[:tc_rows].reshape(g, 1, _TC_BLOCK)
    tc_partials = pl.pallas_call(
        _loss_block_kernel,
        grid=(g,),
        in_specs=[
            pl.BlockSpec((_TC_BLOCK, v), lambda i: (i, 0)),
            pl.BlockSpec((1, 1, _TC_BLOCK), lambda i: (i, 0, 0)),
        ],
        out_specs=pl.BlockSpec((1, 1, 1), lambda i: (i, 0, 0)),
        out_shape=jax.ShapeDtypeStruct((g, 1, 1), jnp.float32),
        compiler_params=pltpu.CompilerParams(
            dimension_semantics=("parallel",),
            vmem_limit_bytes=128 * 1024 * 1024,
        ),
    )(x, tgt3)
    sc_partials = _make_sc_sum(tc_rows)(x, tgt)
    const = n * ((v - 1) * _EPS * math.log(_EPS) + _CONF * math.log(_CONF))
    return jnp.float32(const) + jnp.sum(tc_partials) + jnp.sum(sc_partials)
